# Initial kernel scaffold; baseline (speedup 1.0000x reference)
import jax
import jax.numpy as jnp
from jax.experimental import pallas as pl


def kernel(output, target):
    raise NotImplementedError("write your pallas kernel here")


# speedup vs baseline: 1.1010x; 1.1010x over previous
import functools

import jax
import jax.numpy as jnp
from jax import lax
from jax.experimental import pallas as pl
from jax.experimental.pallas import tpu as pltpu
from jax.experimental.pallas import tpu_sc as plsc

SC_LANES = 16
N_WORKERS = 32

TOPK_SMALL = 1
TOPK_LARGE = 5

ROW_BLK = 256
COL_BLK = 2048


def _sc_gather_body(bpw, n_cols, table128, target, tv128,
                    tgt_v, idx_v, rows_v, sem):
    wid = lax.axis_index("s") * 2 + lax.axis_index("c")
    base = wid * bpw
    pltpu.sync_copy(target.at[pl.ds(base, bpw)], tgt_v)
    for c in range(bpw // SC_LANES):
        row16 = base + c * SC_LANES + lax.iota(jnp.int32, SC_LANES)
        t16 = tgt_v[pl.ds(c * SC_LANES, SC_LANES)]
        idx_v[pl.ds(c * SC_LANES, SC_LANES)] = lax.shift_right_logical(
            row16 * n_cols + t16, 7)
    pltpu.async_copy(table128.at[idx_v], rows_v, sem).wait()
    pltpu.sync_copy(rows_v, tv128.at[pl.ds(base, bpw)])


def _tc_count_body(n_rows, n_cols, cb_last, tv128_ref, tgt_ref, x_ref,
                   out_ref, acc_ref, tvs_ref):
    rb = pl.program_id(0)
    cb = pl.program_id(1)
    tg = tgt_ref[...]

    @pl.when(cb == 0)
    def _():
        acc_ref[...] = jnp.zeros_like(acc_ref)
        row = rb * ROW_BLK + lax.broadcasted_iota(
            jnp.int32, (ROW_BLK, 1), 0)
        tlane = jnp.bitwise_and(row * (n_cols % 128) + tg, 127)
        lane = lax.broadcasted_iota(jnp.int32, (ROW_BLK, 128), 1)
        tvs_ref[...] = jnp.sum(
            jnp.where(lane == tlane, tv128_ref[...], 0.0),
            axis=1, keepdims=True)

    x = x_ref[...]
    tv = tvs_ref[...]
    col = cb * COL_BLK + lax.broadcasted_iota(
        jnp.int32, (ROW_BLK, COL_BLK), 1)
    valid = col < n_cols
    better = ((x > tv) | ((x == tv) & (col < tg))) & valid
    acc_ref[...] += jnp.sum(
        better.astype(jnp.float32), axis=1, keepdims=True)

    @pl.when(cb == cb_last)
    def _():
        a = acc_ref[...]
        inv_b = 1.0 / n_rows
        s1 = jnp.sum((a < TOPK_SMALL).astype(jnp.float32)) * inv_b
        s5 = jnp.sum((a < TOPK_LARGE).astype(jnp.float32)) * inv_b
        lane = lax.broadcasted_iota(jnp.int32, (1, 128), 1)
        contrib = (jnp.where(lane == 0, s1, 0.0)
                   + jnp.where(lane == 1, s5, 0.0))

        @pl.when(rb == 0)
        def _():
            out_ref[...] = contrib

        @pl.when(rb > 0)
        def _():
            out_ref[...] += contrib


def kernel(output, target):
    n_rows, n_cols = output.shape
    tgt = target.astype(jnp.int32)

    table128 = output.reshape(n_rows * n_cols // 128, 128)
    bpw = n_rows // N_WORKERS
    mesh = plsc.VectorSubcoreMesh(core_axis_name="c", subcore_axis_name="s")
    tv128 = pl.kernel(
        functools.partial(_sc_gather_body, bpw, n_cols),
        out_type=jax.ShapeDtypeStruct((n_rows, 128), jnp.float32),
        mesh=mesh,
        scratch_types=[
            pltpu.VMEM((bpw,), jnp.int32),
            pltpu.VMEM((bpw,), jnp.int32),
            pltpu.VMEM((bpw, 128), jnp.float32),
            pltpu.SemaphoreType.DMA,
        ],
    )(table128, tgt)

    rb_n = n_rows // ROW_BLK
    cb_n = (n_cols + COL_BLK - 1) // COL_BLK
    out = pl.pallas_call(
        functools.partial(_tc_count_body, n_rows, n_cols, cb_n - 1),
        grid=(rb_n, cb_n),
        in_specs=[
            pl.BlockSpec((ROW_BLK, 128), lambda rb, cb: (rb, 0)),
            pl.BlockSpec((ROW_BLK, 1), lambda rb, cb: (rb, 0)),
            pl.BlockSpec((ROW_BLK, COL_BLK), lambda rb, cb: (rb, cb)),
        ],
        out_specs=pl.BlockSpec((1, 128), lambda rb, cb: (0, 0)),
        out_shape=jax.ShapeDtypeStruct((1, 128), jnp.float32),
        scratch_shapes=[
            pltpu.VMEM((ROW_BLK, 1), jnp.float32),
            pltpu.VMEM((ROW_BLK, 1), jnp.float32),
        ],
    )(tv128, tgt.reshape(n_rows, 1), output)
    return out[0, :2]


# speedup vs baseline: 1.9378x; 1.7599x over previous
import functools

import jax
import jax.numpy as jnp
from jax import lax
from jax.experimental import pallas as pl
from jax.experimental.pallas import tpu as pltpu

TOPK_SMALL = 1
TOPK_LARGE = 5

GROWS = 8
ROW_BLK = 256
COL_BLK = 2048


def _gather_body(tref, *refs):
    xs = refs[:GROWS]
    win_ref = refs[GROWS]
    for r in range(GROWS):
        win_ref[pl.ds(r, 1), :] = xs[r][pl.ds(r, 1), :]


def _count_body(n_rows, n_cols, cb_last, win_ref, tgt_ref, x_ref,
                out_ref, acc_ref, tvs_ref):
    rb = pl.program_id(0)
    cb = pl.program_id(1)
    tg = tgt_ref[...]

    @pl.when(cb == 0)
    def _():
        acc_ref[...] = jnp.zeros_like(acc_ref)
        tlane = jnp.bitwise_and(tg, 127)
        lane = lax.broadcasted_iota(jnp.int32, (ROW_BLK, 128), 1)
        tvs_ref[...] = jnp.sum(
            jnp.where(lane == tlane, win_ref[...], 0.0),
            axis=1, keepdims=True)

    x = x_ref[...]
    tv = tvs_ref[...]
    col = cb * COL_BLK + lax.broadcasted_iota(
        jnp.int32, (ROW_BLK, COL_BLK), 1)
    valid = col < n_cols
    better = ((x > tv) | ((x == tv) & (col < tg))) & valid
    acc_ref[...] += jnp.sum(
        better.astype(jnp.float32), axis=1, keepdims=True)

    @pl.when(cb == cb_last)
    def _():
        a = acc_ref[...]
        inv_b = 1.0 / n_rows
        s1 = jnp.sum((a < TOPK_SMALL).astype(jnp.float32)) * inv_b
        s5 = jnp.sum((a < TOPK_LARGE).astype(jnp.float32)) * inv_b
        lane = lax.broadcasted_iota(jnp.int32, (1, 128), 1)
        contrib = (jnp.where(lane == 0, s1, 0.0)
                   + jnp.where(lane == 1, s5, 0.0))

        @pl.when(rb == 0)
        def _():
            out_ref[...] = contrib

        @pl.when(rb > 0)
        def _():
            out_ref[...] += contrib


def _window_spec(r):
    return pl.BlockSpec(
        (GROWS, 128), lambda i, tref: (i, tref[GROWS * i + r] // 128))


def kernel(output, target):
    n_rows, n_cols = output.shape
    tgt = target.astype(jnp.int32)

    win = pl.pallas_call(
        _gather_body,
        grid_spec=pltpu.PrefetchScalarGridSpec(
            num_scalar_prefetch=1,
            grid=(n_rows // GROWS,),
            in_specs=[_window_spec(r) for r in range(GROWS)],
            out_specs=pl.BlockSpec((GROWS, 128), lambda i, tref: (i, 0)),
        ),
        out_shape=jax.ShapeDtypeStruct((n_rows, 128), jnp.float32),
    )(tgt, *([output] * GROWS))

    rb_n = n_rows // ROW_BLK
    cb_n = (n_cols + COL_BLK - 1) // COL_BLK
    out = pl.pallas_call(
        functools.partial(_count_body, n_rows, n_cols, cb_n - 1),
        grid=(rb_n, cb_n),
        in_specs=[
            pl.BlockSpec((ROW_BLK, 128), lambda rb, cb: (rb, 0)),
            pl.BlockSpec((ROW_BLK, 1), lambda rb, cb: (rb, 0)),
            pl.BlockSpec((ROW_BLK, COL_BLK), lambda rb, cb: (rb, cb)),
        ],
        out_specs=pl.BlockSpec((1, 128), lambda rb, cb: (0, 0)),
        out_shape=jax.ShapeDtypeStruct((1, 128), jnp.float32),
        scratch_shapes=[
            pltpu.VMEM((ROW_BLK, 1), jnp.float32),
            pltpu.VMEM((ROW_BLK, 1), jnp.float32),
        ],
    )(win, tgt.reshape(n_rows, 1), output)
    return out[0, :2]


# speedup vs baseline: 1.9971x; 1.0306x over previous
import functools

import jax
import jax.numpy as jnp
from jax import lax
from jax.experimental import pallas as pl
from jax.experimental.pallas import tpu as pltpu

TOPK_SMALL = 1
TOPK_LARGE = 5

NWIN = 32
ROW_BLK = 256
COL_BLK = 2048


def _gather_body(tref, *refs):
    xs = refs[:NWIN]
    win_ref = refs[NWIN]
    for r in range(NWIN):
        win_ref[pl.ds(r, 1), :] = xs[r][pl.ds(r % 8, 1), :]


def _count_body(n_rows, n_cols, cb_last, win_ref, tgt_ref, x_ref,
                out_ref, acc_ref, tvs_ref):
    rb = pl.program_id(0)
    cb = pl.program_id(1)
    tg = tgt_ref[...]

    @pl.when(cb == 0)
    def _():
        acc_ref[...] = jnp.zeros_like(acc_ref)
        tlane = jnp.bitwise_and(tg, 127)
        lane = lax.broadcasted_iota(jnp.int32, (ROW_BLK, 128), 1)
        tvs_ref[...] = jnp.sum(
            jnp.where(lane == tlane, win_ref[...], 0.0),
            axis=1, keepdims=True)

    x = x_ref[...]
    tv = tvs_ref[...]
    col = cb * COL_BLK + lax.broadcasted_iota(
        jnp.int32, (ROW_BLK, COL_BLK), 1)
    valid = col < n_cols
    better = ((x > tv) | ((x == tv) & (col < tg))) & valid
    acc_ref[...] += jnp.sum(
        better.astype(jnp.float32), axis=1, keepdims=True)

    @pl.when(cb == cb_last)
    def _():
        a = acc_ref[...]
        inv_b = 1.0 / n_rows
        s1 = jnp.sum((a < TOPK_SMALL).astype(jnp.float32)) * inv_b
        s5 = jnp.sum((a < TOPK_LARGE).astype(jnp.float32)) * inv_b
        lane = lax.broadcasted_iota(jnp.int32, (1, 128), 1)
        contrib = (jnp.where(lane == 0, s1, 0.0)
                   + jnp.where(lane == 1, s5, 0.0))

        @pl.when(rb == 0)
        def _():
            out_ref[...] = contrib

        @pl.when(rb > 0)
        def _():
            out_ref[...] += contrib


def _window_spec(r):
    return pl.BlockSpec(
        (8, 128),
        lambda i, tref: ((NWIN * i + r) // 8, tref[NWIN * i + r] // 128))


def kernel(output, target):
    n_rows, n_cols = output.shape
    tgt = target.astype(jnp.int32)

    win = pl.pallas_call(
        _gather_body,
        grid_spec=pltpu.PrefetchScalarGridSpec(
            num_scalar_prefetch=1,
            grid=(n_rows // NWIN,),
            in_specs=[_window_spec(r) for r in range(NWIN)],
            out_specs=pl.BlockSpec((NWIN, 128), lambda i, tref: (i, 0)),
        ),
        out_shape=jax.ShapeDtypeStruct((n_rows, 128), jnp.float32),
    )(tgt, *([output] * NWIN))

    rb_n = n_rows // ROW_BLK
    cb_n = (n_cols + COL_BLK - 1) // COL_BLK
    out = pl.pallas_call(
        functools.partial(_count_body, n_rows, n_cols, cb_n - 1),
        grid=(rb_n, cb_n),
        in_specs=[
            pl.BlockSpec((ROW_BLK, 128), lambda rb, cb: (rb, 0)),
            pl.BlockSpec((ROW_BLK, 1), lambda rb, cb: (rb, 0)),
            pl.BlockSpec((ROW_BLK, COL_BLK), lambda rb, cb: (rb, cb)),
        ],
        out_specs=pl.BlockSpec((1, 128), lambda rb, cb: (0, 0)),
        out_shape=jax.ShapeDtypeStruct((1, 128), jnp.float32),
        scratch_shapes=[
            pltpu.VMEM((ROW_BLK, 1), jnp.float32),
            pltpu.VMEM((ROW_BLK, 1), jnp.float32),
        ],
    )(win, tgt.reshape(n_rows, 1), output)
    return out[0, :2]


# speedup vs baseline: 4.5263x; 2.2664x over previous
import functools

import jax
import jax.numpy as jnp
from jax import lax
from jax.experimental import pallas as pl
from jax.experimental.pallas import tpu as pltpu

TOPK_SMALL = 1
TOPK_LARGE = 5

NWIN = 128
BAT_BLK = 256
VOC_BLK = 2048


def _gather_body(tref, *refs):
    xs = refs[:NWIN]
    win_ref = refs[NWIN]
    lane = lax.broadcasted_iota(jnp.int32, (8, NWIN), 1)
    acc = jnp.zeros((8, NWIN), jnp.float32)
    for r in range(NWIN):
        acc = acc + jnp.where(lane == r, xs[r][...], 0.0)
    win_ref[...] = acc


def _win_spec(r):
    return pl.BlockSpec(
        (8, NWIN), lambda i, tref: (tref[NWIN * i + r] // 8, i))


def _count_body(n_bat, n_voc, cb_last, win_ref, tgt_ref, x_ref,
                out_ref, acc_ref, tvs_ref):
    rb = pl.program_id(0)
    cb = pl.program_id(1)
    tg = tgt_ref[...]

    @pl.when(cb == 0)
    def _():
        acc_ref[...] = jnp.zeros_like(acc_ref)
        srow = lax.broadcasted_iota(jnp.int32, (8, BAT_BLK), 0)
        sel = srow == jnp.bitwise_and(tg, 7)
        tvs_ref[...] = jnp.sum(
            jnp.where(sel, win_ref[...], 0.0), axis=0, keepdims=True)

    x = x_ref[...]
    tv = tvs_ref[...]
    vidx = cb * VOC_BLK + lax.broadcasted_iota(
        jnp.int32, (VOC_BLK, BAT_BLK), 0)
    valid = vidx < n_voc
    better = ((x > tv) | ((x == tv) & (vidx < tg))) & valid
    acc_ref[...] += jnp.sum(
        better.astype(jnp.float32), axis=0, keepdims=True)

    @pl.when(cb == cb_last)
    def _():
        a = acc_ref[...]
        inv_b = 1.0 / n_bat
        s1 = jnp.sum((a < TOPK_SMALL).astype(jnp.float32)) * inv_b
        s5 = jnp.sum((a < TOPK_LARGE).astype(jnp.float32)) * inv_b
        lane = lax.broadcasted_iota(jnp.int32, (1, 128), 1)
        contrib = (jnp.where(lane == 0, s1, 0.0)
                   + jnp.where(lane == 1, s5, 0.0))

        @pl.when(rb == 0)
        def _():
            out_ref[...] = contrib

        @pl.when(rb > 0)
        def _():
            out_ref[...] += contrib


def kernel(output, target):
    n_bat, n_voc = output.shape
    tgt = target.astype(jnp.int32)
    xt = output.T

    win = pl.pallas_call(
        _gather_body,
        grid_spec=pltpu.PrefetchScalarGridSpec(
            num_scalar_prefetch=1,
            grid=(n_bat // NWIN,),
            in_specs=[_win_spec(r) for r in range(NWIN)],
            out_specs=pl.BlockSpec((8, NWIN), lambda i, tref: (0, i)),
        ),
        out_shape=jax.ShapeDtypeStruct((8, n_bat), jnp.float32),
    )(tgt, *([xt] * NWIN))

    rb_n = n_bat // BAT_BLK
    cb_n = (n_voc + VOC_BLK - 1) // VOC_BLK
    out = pl.pallas_call(
        functools.partial(_count_body, n_bat, n_voc, cb_n - 1),
        grid=(rb_n, cb_n),
        in_specs=[
            pl.BlockSpec((8, BAT_BLK), lambda rb, cb: (0, rb)),
            pl.BlockSpec((1, BAT_BLK), lambda rb, cb: (0, rb)),
            pl.BlockSpec((VOC_BLK, BAT_BLK), lambda rb, cb: (cb, rb)),
        ],
        out_specs=pl.BlockSpec((1, 128), lambda rb, cb: (0, 0)),
        out_shape=jax.ShapeDtypeStruct((1, 128), jnp.float32),
        scratch_shapes=[
            pltpu.VMEM((1, BAT_BLK), jnp.float32),
            pltpu.VMEM((1, BAT_BLK), jnp.float32),
        ],
    )(win, tgt.reshape(1, n_bat), xt)
    return out[0, :2]


# speedup vs baseline: 5.3892x; 1.1906x over previous
import functools

import jax
import jax.numpy as jnp
from jax import lax
from jax.experimental import pallas as pl
from jax.experimental.pallas import tpu as pltpu

TOPK_SMALL = 1
TOPK_LARGE = 5

NWIN = 128
BAT_BLK = 256
VOC_BLK = 4096


def _gather_body(tref, *refs):
    xs = refs[:NWIN]
    win_ref = refs[NWIN]
    lane = lax.broadcasted_iota(jnp.int32, (8, NWIN), 1)
    acc = jnp.zeros((8, NWIN), jnp.float32)
    for r in range(NWIN):
        acc = acc + jnp.where(lane == r, xs[r][...], 0.0)
    win_ref[...] = acc


def _win_spec(r):
    return pl.BlockSpec(
        (8, NWIN), lambda i, tref: (tref[NWIN * i + r] // 8, i))


def _count_body(n_bat, n_voc, cb_last, win_ref, tgt_ref, x_ref,
                out_ref, acc_ref, tvs_ref):
    rb = pl.program_id(0)
    cb = pl.program_id(1)
    tg = tgt_ref[...]

    @pl.when(cb == 0)
    def _():
        acc_ref[...] = jnp.zeros_like(acc_ref)
        srow = lax.broadcasted_iota(jnp.int32, (8, BAT_BLK), 0)
        sel = srow == jnp.bitwise_and(tg, 7)
        tvs_ref[...] = jnp.sum(
            jnp.where(sel, win_ref[...], 0.0), axis=0, keepdims=True)

    x = x_ref[...]
    tv = tvs_ref[...]
    vidx = cb * VOC_BLK + lax.broadcasted_iota(
        jnp.int32, (VOC_BLK, BAT_BLK), 0)
    valid = vidx < n_voc
    better = ((x > tv) | ((x == tv) & (vidx < tg))) & valid
    acc_ref[...] += jnp.sum(
        better.astype(jnp.float32), axis=0, keepdims=True)

    @pl.when(cb == cb_last)
    def _():
        a = acc_ref[...]
        inv_b = 1.0 / n_bat
        s1 = jnp.sum((a < TOPK_SMALL).astype(jnp.float32)) * inv_b
        s5 = jnp.sum((a < TOPK_LARGE).astype(jnp.float32)) * inv_b
        lane = lax.broadcasted_iota(jnp.int32, (1, 128), 1)
        contrib = (jnp.where(lane == 0, s1, 0.0)
                   + jnp.where(lane == 1, s5, 0.0))

        @pl.when(rb == 0)
        def _():
            out_ref[...] = contrib

        @pl.when(rb > 0)
        def _():
            out_ref[...] += contrib


def kernel(output, target):
    n_bat, n_voc = output.shape
    tgt = target.astype(jnp.int32)
    xt = output.T

    win = pl.pallas_call(
        _gather_body,
        grid_spec=pltpu.PrefetchScalarGridSpec(
            num_scalar_prefetch=1,
            grid=(n_bat // NWIN,),
            in_specs=[_win_spec(r) for r in range(NWIN)],
            out_specs=pl.BlockSpec((8, NWIN), lambda i, tref: (0, i)),
        ),
        out_shape=jax.ShapeDtypeStruct((8, n_bat), jnp.float32),
    )(tgt, *([xt] * NWIN))

    rb_n = n_bat // BAT_BLK
    cb_n = (n_voc + VOC_BLK - 1) // VOC_BLK
    out = pl.pallas_call(
        functools.partial(_count_body, n_bat, n_voc, cb_n - 1),
        grid=(rb_n, cb_n),
        in_specs=[
            pl.BlockSpec((8, BAT_BLK), lambda rb, cb: (0, rb)),
            pl.BlockSpec((1, BAT_BLK), lambda rb, cb: (0, rb)),
            pl.BlockSpec((VOC_BLK, BAT_BLK), lambda rb, cb: (cb, rb)),
        ],
        out_specs=pl.BlockSpec((1, 128), lambda rb, cb: (0, 0)),
        out_shape=jax.ShapeDtypeStruct((1, 128), jnp.float32),
        scratch_shapes=[
            pltpu.VMEM((1, BAT_BLK), jnp.float32),
            pltpu.VMEM((1, BAT_BLK), jnp.float32),
        ],
    )(win, tgt.reshape(1, n_bat), xt)
    return out[0, :2]


# speedup vs baseline: 5.6331x; 1.0453x over previous
import functools

import jax
import jax.numpy as jnp
from jax import lax
from jax.experimental import pallas as pl
from jax.experimental.pallas import tpu as pltpu

TOPK_SMALL = 1
TOPK_LARGE = 5

NWIN = 128
BAT_BLK = 256
VOC_BLK = 8192


def _gather_body(tref, *refs):
    xs = refs[:NWIN]
    win_ref = refs[NWIN]
    lane = lax.broadcasted_iota(jnp.int32, (8, NWIN), 1)
    acc = jnp.zeros((8, NWIN), jnp.float32)
    for r in range(NWIN):
        acc = acc + jnp.where(lane == r, xs[r][...], 0.0)
    win_ref[...] = acc


def _win_spec(r):
    return pl.BlockSpec(
        (8, NWIN), lambda i, tref: (tref[NWIN * i + r] // 8, i))


def _count_body(n_bat, n_voc, cb_last, win_ref, tgt_ref, x_ref,
                out_ref, acc_ref, tvs_ref):
    rb = pl.program_id(0)
    cb = pl.program_id(1)
    tg = tgt_ref[...]

    @pl.when(cb == 0)
    def _():
        acc_ref[...] = jnp.zeros_like(acc_ref)
        srow = lax.broadcasted_iota(jnp.int32, (8, BAT_BLK), 0)
        sel = srow == jnp.bitwise_and(tg, 7)
        tvs_ref[...] = jnp.sum(
            jnp.where(sel, win_ref[...], 0.0), axis=0, keepdims=True)

    x = x_ref[...]
    tv = tvs_ref[...]
    vidx = cb * VOC_BLK + lax.broadcasted_iota(
        jnp.int32, (VOC_BLK, BAT_BLK), 0)
    valid = vidx < n_voc
    better = ((x > tv) | ((x == tv) & (vidx < tg))) & valid
    acc_ref[...] += jnp.sum(
        better.astype(jnp.float32), axis=0, keepdims=True)

    @pl.when(cb == cb_last)
    def _():
        a = acc_ref[...]
        inv_b = 1.0 / n_bat
        s1 = jnp.sum((a < TOPK_SMALL).astype(jnp.float32)) * inv_b
        s5 = jnp.sum((a < TOPK_LARGE).astype(jnp.float32)) * inv_b
        lane = lax.broadcasted_iota(jnp.int32, (1, 128), 1)
        contrib = (jnp.where(lane == 0, s1, 0.0)
                   + jnp.where(lane == 1, s5, 0.0))

        @pl.when(rb == 0)
        def _():
            out_ref[...] = contrib

        @pl.when(rb > 0)
        def _():
            out_ref[...] += contrib


def kernel(output, target):
    n_bat, n_voc = output.shape
    tgt = target.astype(jnp.int32)
    xt = output.T

    win = pl.pallas_call(
        _gather_body,
        grid_spec=pltpu.PrefetchScalarGridSpec(
            num_scalar_prefetch=1,
            grid=(n_bat // NWIN,),
            in_specs=[_win_spec(r) for r in range(NWIN)],
            out_specs=pl.BlockSpec((8, NWIN), lambda i, tref: (0, i)),
        ),
        out_shape=jax.ShapeDtypeStruct((8, n_bat), jnp.float32),
    )(tgt, *([xt] * NWIN))

    rb_n = n_bat // BAT_BLK
    cb_n = (n_voc + VOC_BLK - 1) // VOC_BLK
    out = pl.pallas_call(
        functools.partial(_count_body, n_bat, n_voc, cb_n - 1),
        grid=(rb_n, cb_n),
        in_specs=[
            pl.BlockSpec((8, BAT_BLK), lambda rb, cb: (0, rb)),
            pl.BlockSpec((1, BAT_BLK), lambda rb, cb: (0, rb)),
            pl.BlockSpec((VOC_BLK, BAT_BLK), lambda rb, cb: (cb, rb)),
        ],
        out_specs=pl.BlockSpec((1, 128), lambda rb, cb: (0, 0)),
        out_shape=jax.ShapeDtypeStruct((1, 128), jnp.float32),
        scratch_shapes=[
            pltpu.VMEM((1, BAT_BLK), jnp.float32),
            pltpu.VMEM((1, BAT_BLK), jnp.float32),
        ],
    )(win, tgt.reshape(1, n_bat), xt)
    return out[0, :2]


# speedup vs baseline: 5.8302x; 1.0350x over previous
import functools

import jax
import jax.numpy as jnp
from jax import lax
from jax.experimental import pallas as pl
from jax.experimental.pallas import tpu as pltpu

TOPK_SMALL = 1
TOPK_LARGE = 5

NWIN = 128
BAT_BLK = 256
VOC_BLK = 12800


def _gather_body(tref, *refs):
    xs = refs[:NWIN]
    win_ref = refs[NWIN]
    lane = lax.broadcasted_iota(jnp.int32, (8, NWIN), 1)
    acc = jnp.zeros((8, NWIN), jnp.float32)
    for r in range(NWIN):
        acc = acc + jnp.where(lane == r, xs[r][...], 0.0)
    win_ref[...] = acc


def _win_spec(r):
    return pl.BlockSpec(
        (8, NWIN), lambda i, tref: (tref[NWIN * i + r] // 8, i))


def _count_body(n_bat, n_voc, cb_last, win_ref, tgt_ref, x_ref,
                out_ref, acc_ref, tvs_ref):
    rb = pl.program_id(0)
    cb = pl.program_id(1)
    tg = tgt_ref[...]

    @pl.when(cb == 0)
    def _():
        acc_ref[...] = jnp.zeros_like(acc_ref)
        srow = lax.broadcasted_iota(jnp.int32, (8, BAT_BLK), 0)
        sel = srow == jnp.bitwise_and(tg, 7)
        tvs_ref[...] = jnp.sum(
            jnp.where(sel, win_ref[...], 0.0), axis=0, keepdims=True)

    x = x_ref[...]
    tv = tvs_ref[...]
    vidx = cb * VOC_BLK + lax.broadcasted_iota(
        jnp.int32, (VOC_BLK, BAT_BLK), 0)
    valid = vidx < n_voc
    better = ((x > tv) | ((x == tv) & (vidx < tg))) & valid
    acc_ref[...] += jnp.sum(
        better.astype(jnp.float32), axis=0, keepdims=True)

    @pl.when(cb == cb_last)
    def _():
        a = acc_ref[...]
        inv_b = 1.0 / n_bat
        s1 = jnp.sum((a < TOPK_SMALL).astype(jnp.float32)) * inv_b
        s5 = jnp.sum((a < TOPK_LARGE).astype(jnp.float32)) * inv_b
        lane = lax.broadcasted_iota(jnp.int32, (1, 128), 1)
        contrib = (jnp.where(lane == 0, s1, 0.0)
                   + jnp.where(lane == 1, s5, 0.0))

        @pl.when(rb == 0)
        def _():
            out_ref[...] = contrib

        @pl.when(rb > 0)
        def _():
            out_ref[...] += contrib


def kernel(output, target):
    n_bat, n_voc = output.shape
    tgt = target.astype(jnp.int32)
    xt = output.T

    win = pl.pallas_call(
        _gather_body,
        grid_spec=pltpu.PrefetchScalarGridSpec(
            num_scalar_prefetch=1,
            grid=(n_bat // NWIN,),
            in_specs=[_win_spec(r) for r in range(NWIN)],
            out_specs=pl.BlockSpec((8, NWIN), lambda i, tref: (0, i)),
        ),
        out_shape=jax.ShapeDtypeStruct((8, n_bat), jnp.float32),
    )(tgt, *([xt] * NWIN))

    rb_n = n_bat // BAT_BLK
    cb_n = (n_voc + VOC_BLK - 1) // VOC_BLK
    out = pl.pallas_call(
        functools.partial(_count_body, n_bat, n_voc, cb_n - 1),
        grid=(rb_n, cb_n),
        in_specs=[
            pl.BlockSpec((8, BAT_BLK), lambda rb, cb: (0, rb)),
            pl.BlockSpec((1, BAT_BLK), lambda rb, cb: (0, rb)),
            pl.BlockSpec((VOC_BLK, BAT_BLK), lambda rb, cb: (cb, rb)),
        ],
        out_specs=pl.BlockSpec((1, 128), lambda rb, cb: (0, 0)),
        out_shape=jax.ShapeDtypeStruct((1, 128), jnp.float32),
        scratch_shapes=[
            pltpu.VMEM((1, BAT_BLK), jnp.float32),
            pltpu.VMEM((1, BAT_BLK), jnp.float32),
        ],
    )(win, tgt.reshape(1, n_bat), xt)
    return out[0, :2]
